# baseline (speedup 1.0000x reference)
import jax
import jax.numpy as jnp
from jax import lax
from jax.experimental import pallas as pl
from jax.experimental.pallas import tpu as pltpu
from jax.experimental.pallas import tpu_sc as plsc

BATCH = 512
FEATURE_NUM = 39
EMB = 8
N_IDS = BATCH * FEATURE_NUM

NC = 2
NS = 16
NW = NC * NS
LANES = 16
PER_W = N_IDS // NW


def _sc_gather_body(val_hbm, idx_hbm, table_hbm, out_hbm,
                    idx_v, cid_v, rows_v, sem):
    wid = lax.axis_index("s") * NC + lax.axis_index("c")
    base = wid * PER_W
    pltpu.sync_copy(idx_hbm.at[pl.ds(base, PER_W)], idx_v)
    pltpu.async_copy(val_hbm.at[idx_v], cid_v, sem).wait()
    pltpu.async_copy(table_hbm.at[cid_v], rows_v, sem).wait()
    pltpu.sync_copy(rows_v, out_hbm.at[pl.ds(base, PER_W)])


def _sc_gather(ft_sparse_val, ft_sparse_idx, emb_table):
    mesh = plsc.VectorSubcoreMesh(core_axis_name="c", subcore_axis_name="s")
    fn = pl.kernel(
        _sc_gather_body, mesh=mesh,
        out_type=jax.ShapeDtypeStruct((N_IDS, EMB), jnp.float32),
        scratch_types=[
            pltpu.VMEM((PER_W,), jnp.int32),
            pltpu.VMEM((PER_W,), jnp.int32),
            pltpu.VMEM((PER_W, EMB), jnp.float32),
            pltpu.SemaphoreType.DMA,
        ],
        compiler_params=pltpu.CompilerParams(use_tc_tiling_on_sc=False),
    )
    return fn(ft_sparse_val, ft_sparse_idx, emb_table)


def _mlp_body(x_ref, w0_ref, b0_ref, w1_ref, b1_ref, w2_ref, b2_ref, y_ref,
              logits_ref, predict_ref, loss_ref):
    x = x_ref[...]
    h = jnp.dot(x, w0_ref[...], preferred_element_type=jnp.float32)
    h = jnp.maximum(h + b0_ref[...], 0.0)
    h = jnp.dot(h, w1_ref[...], preferred_element_type=jnp.float32)
    h = jnp.maximum(h + b1_ref[...], 0.0)
    z = jnp.dot(h, w2_ref[...], preferred_element_type=jnp.float32) + b2_ref[...]
    logits_ref[...] = z
    predict_ref[...] = jax.nn.sigmoid(z)
    y = y_ref[...]
    loss_vec = jnp.maximum(z, 0.0) - z * y + jnp.log1p(jnp.exp(-jnp.abs(z)))
    loss_ref[0, 0] = jnp.sum(loss_vec) * (1.0 / BATCH)


def _mlp(x, W0, b0, W1, b1, W2, b2, labels):
    logits, predict, loss = pl.pallas_call(
        _mlp_body,
        out_shape=(
            jax.ShapeDtypeStruct((BATCH, 1), jnp.float32),
            jax.ShapeDtypeStruct((BATCH, 1), jnp.float32),
            jax.ShapeDtypeStruct((1, 1), jnp.float32),
        ),
        out_specs=(
            pl.BlockSpec(memory_space=pltpu.VMEM),
            pl.BlockSpec(memory_space=pltpu.VMEM),
            pl.BlockSpec(memory_space=pltpu.SMEM),
        ),
    )(x, W0, b0.reshape(1, -1), W1, b1.reshape(1, -1), W2,
      b2.reshape(1, 1), labels)
    return logits, predict, loss[0, 0]


def kernel(labels, ft_sparse_val, ft_sparse_idx, emb_table, W0, b0, W1, b1,
           W2, b2):
    rows = _sc_gather(ft_sparse_val, ft_sparse_idx, emb_table)
    x = rows.reshape(BATCH, FEATURE_NUM * EMB)
    logits, predict, loss = _mlp(x, W0, b0, W1, b1, W2, b2, labels)
    size = jnp.array(emb_table.shape[0], dtype=jnp.int32)
    return (labels, logits, predict, loss, size)


# speedup vs baseline: 2.4118x; 2.4118x over previous
import jax
import jax.numpy as jnp
from jax import lax
from jax.experimental import pallas as pl
from jax.experimental.pallas import tpu as pltpu
from jax.experimental.pallas import tpu_sc as plsc

BATCH = 512
FEATURE_NUM = 39
EMB = 8
N_IDS = BATCH * FEATURE_NUM

NC = 2
NS = 16
NW = NC * NS
LANES = 16
PER_W = N_IDS // NW


def _sc_gather_body(val_hbm, idx_hbm, c0, c1, c2, c3, c4, c5, c6, c7,
                    out_hbm, idx_v, cid_v, rows_v, sem):
    wid = lax.axis_index("s") * NC + lax.axis_index("c")
    base = wid * PER_W
    cols = (c0, c1, c2, c3, c4, c5, c6, c7)
    pltpu.sync_copy(idx_hbm.at[pl.ds(base, PER_W)], idx_v)
    pltpu.async_copy(val_hbm.at[idx_v], cid_v, sem).wait()
    copies = [pltpu.async_copy(cols[e].at[cid_v],
                               rows_v.at[pl.ds(e * PER_W, PER_W)], sem)
              for e in range(EMB)]
    for c in copies:
        c.wait()
    pltpu.sync_copy(rows_v, out_hbm.at[pl.ds(wid * EMB * PER_W, EMB * PER_W)])


def _sc_gather(ft_sparse_val, ft_sparse_idx, emb_table):
    mesh = plsc.VectorSubcoreMesh(core_axis_name="c", subcore_axis_name="s")
    fn = pl.kernel(
        _sc_gather_body, mesh=mesh,
        out_type=jax.ShapeDtypeStruct((NW * EMB * PER_W,), jnp.float32),
        scratch_types=[
            pltpu.VMEM((PER_W,), jnp.int32),
            pltpu.VMEM((PER_W,), jnp.int32),
            pltpu.VMEM((EMB * PER_W,), jnp.float32),
            pltpu.SemaphoreType.DMA,
        ],
    )
    cols = [emb_table[:, e] for e in range(EMB)]
    return fn(ft_sparse_val, ft_sparse_idx, *cols)


def _mlp_body(x_ref, w0_ref, b0_ref, w1_ref, b1_ref, w2_ref, b2_ref, y_ref,
              logits_ref, predict_ref, loss_ref):
    x = x_ref[...]
    h = jnp.dot(x, w0_ref[...], preferred_element_type=jnp.float32)
    h = jnp.maximum(h + b0_ref[...], 0.0)
    h = jnp.dot(h, w1_ref[...], preferred_element_type=jnp.float32)
    h = jnp.maximum(h + b1_ref[...], 0.0)
    z = jnp.dot(h, w2_ref[...], preferred_element_type=jnp.float32) + b2_ref[...]
    logits_ref[...] = z
    predict_ref[...] = jax.nn.sigmoid(z)
    y = y_ref[...]
    loss_vec = jnp.maximum(z, 0.0) - z * y + jnp.log1p(jnp.exp(-jnp.abs(z)))
    loss_ref[0, 0] = jnp.sum(loss_vec) * (1.0 / BATCH)


def _mlp(x, W0, b0, W1, b1, W2, b2, labels):
    logits, predict, loss = pl.pallas_call(
        _mlp_body,
        out_shape=(
            jax.ShapeDtypeStruct((BATCH, 1), jnp.float32),
            jax.ShapeDtypeStruct((BATCH, 1), jnp.float32),
            jax.ShapeDtypeStruct((1, 1), jnp.float32),
        ),
        out_specs=(
            pl.BlockSpec(memory_space=pltpu.VMEM),
            pl.BlockSpec(memory_space=pltpu.VMEM),
            pl.BlockSpec(memory_space=pltpu.SMEM),
        ),
    )(x, W0, b0.reshape(1, -1), W1, b1.reshape(1, -1), W2,
      b2.reshape(1, 1), labels)
    return logits, predict, loss[0, 0]


def kernel(labels, ft_sparse_val, ft_sparse_idx, emb_table, W0, b0, W1, b1,
           W2, b2):
    out1d = _sc_gather(ft_sparse_val, ft_sparse_idx, emb_table)
    x = (out1d.reshape(NW, EMB, PER_W).transpose(0, 2, 1)
         .reshape(BATCH, FEATURE_NUM * EMB))
    logits, predict, loss = _mlp(x, W0, b0, W1, b1, W2, b2, labels)
    size = jnp.array(emb_table.shape[0], dtype=jnp.int32)
    return (labels, logits, predict, loss, size)


# speedup vs baseline: 5.6847x; 2.3570x over previous
import jax
import jax.numpy as jnp
from jax import lax
from jax.experimental import pallas as pl
from jax.experimental.pallas import tpu as pltpu
from jax.experimental.pallas import tpu_sc as plsc

BATCH = 512
FEATURE_NUM = 39
EMB = 8
VOCAB = 1000000
N_IDS = BATCH * FEATURE_NUM

NC = 2
NS = 16
NW = NC * NS
LANES = 16
PER_W = N_IDS // NW


_XC = 32768


def _extract_body(t_ref, *col_refs):
    x = t_ref[...]
    for e in range(EMB):
        col_refs[e][...] = x[e, :]


def _extract_cols(emb_table):
    grid = (VOCAB + _XC - 1) // _XC
    return pl.pallas_call(
        _extract_body,
        grid=(grid,),
        in_specs=[pl.BlockSpec((EMB, _XC), lambda i: (0, i))],
        out_shape=[jax.ShapeDtypeStruct((VOCAB,), jnp.float32)] * EMB,
        out_specs=[pl.BlockSpec((_XC,), lambda i: (i,))] * EMB,
    )(emb_table.T)


def _sc_gather_body(val_hbm, idx_hbm, c0, c1, c2, c3, c4, c5, c6, c7,
                    out_hbm, idx_v, cid_v, rows_v, sem):
    wid = lax.axis_index("s") * NC + lax.axis_index("c")
    base = wid * PER_W
    cols = (c0, c1, c2, c3, c4, c5, c6, c7)
    pltpu.sync_copy(idx_hbm.at[pl.ds(base, PER_W)], idx_v)
    pltpu.async_copy(val_hbm.at[idx_v], cid_v, sem).wait()
    copies = [pltpu.async_copy(cols[e].at[cid_v],
                               rows_v.at[pl.ds(e * PER_W, PER_W)], sem)
              for e in range(EMB)]
    for c in copies:
        c.wait()
    pltpu.sync_copy(rows_v, out_hbm.at[pl.ds(wid * EMB * PER_W, EMB * PER_W)])


def _sc_gather(ft_sparse_val, ft_sparse_idx, emb_table):
    mesh = plsc.VectorSubcoreMesh(core_axis_name="c", subcore_axis_name="s")
    fn = pl.kernel(
        _sc_gather_body, mesh=mesh,
        out_type=jax.ShapeDtypeStruct((NW * EMB * PER_W,), jnp.float32),
        scratch_types=[
            pltpu.VMEM((PER_W,), jnp.int32),
            pltpu.VMEM((PER_W,), jnp.int32),
            pltpu.VMEM((EMB * PER_W,), jnp.float32),
            pltpu.SemaphoreType.DMA,
        ],
    )
    cols = _extract_cols(emb_table)
    return fn(ft_sparse_val, ft_sparse_idx, *cols)


def _mlp_body(x_ref, w0_ref, b0_ref, w1_ref, b1_ref, w2_ref, b2_ref, y_ref,
              logits_ref, predict_ref, loss_ref):
    x = x_ref[...]
    h = jnp.dot(x, w0_ref[...], preferred_element_type=jnp.float32)
    h = jnp.maximum(h + b0_ref[...], 0.0)
    h = jnp.dot(h, w1_ref[...], preferred_element_type=jnp.float32)
    h = jnp.maximum(h + b1_ref[...], 0.0)
    z = jnp.dot(h, w2_ref[...], preferred_element_type=jnp.float32) + b2_ref[...]
    logits_ref[...] = z
    predict_ref[...] = jax.nn.sigmoid(z)
    y = y_ref[...]
    loss_vec = jnp.maximum(z, 0.0) - z * y + jnp.log1p(jnp.exp(-jnp.abs(z)))
    loss_ref[0, 0] = jnp.sum(loss_vec) * (1.0 / BATCH)


def _mlp(x, W0, b0, W1, b1, W2, b2, labels):
    logits, predict, loss = pl.pallas_call(
        _mlp_body,
        out_shape=(
            jax.ShapeDtypeStruct((BATCH, 1), jnp.float32),
            jax.ShapeDtypeStruct((BATCH, 1), jnp.float32),
            jax.ShapeDtypeStruct((1, 1), jnp.float32),
        ),
        out_specs=(
            pl.BlockSpec(memory_space=pltpu.VMEM),
            pl.BlockSpec(memory_space=pltpu.VMEM),
            pl.BlockSpec(memory_space=pltpu.SMEM),
        ),
    )(x, W0, b0.reshape(1, -1), W1, b1.reshape(1, -1), W2,
      b2.reshape(1, 1), labels)
    return logits, predict, loss[0, 0]


def kernel(labels, ft_sparse_val, ft_sparse_idx, emb_table, W0, b0, W1, b1,
           W2, b2):
    out1d = _sc_gather(ft_sparse_val, ft_sparse_idx, emb_table)
    x = (out1d.reshape(NW, EMB, PER_W).transpose(0, 2, 1)
         .reshape(BATCH, FEATURE_NUM * EMB))
    logits, predict, loss = _mlp(x, W0, b0, W1, b1, W2, b2, labels)
    size = jnp.array(emb_table.shape[0], dtype=jnp.int32)
    return (labels, logits, predict, loss, size)


# speedup vs baseline: 7.4580x; 1.3119x over previous
import jax
import jax.numpy as jnp
from jax import lax
from jax.experimental import pallas as pl
from jax.experimental.pallas import tpu as pltpu
from jax.experimental.pallas import tpu_sc as plsc

BATCH = 512
FEATURE_NUM = 39
EMB = 8
VOCAB = 1000000
N_IDS = BATCH * FEATURE_NUM

NC = 2
NS = 16
NW = NC * NS
LANES = 16
PER_W = N_IDS // NW


_XC = 131072


def _extract_body(t_ref, *col_refs):
    x = t_ref[...]
    for e in range(EMB):
        col_refs[e][...] = x[e, :]


def _extract_cols(emb_table):
    grid = (VOCAB + _XC - 1) // _XC
    return pl.pallas_call(
        _extract_body,
        grid=(grid,),
        in_specs=[pl.BlockSpec((EMB, _XC), lambda i: (0, i))],
        out_shape=[jax.ShapeDtypeStruct((VOCAB,), jnp.float32)] * EMB,
        out_specs=[pl.BlockSpec((_XC,), lambda i: (i,))] * EMB,
    )(emb_table.T)


def _sc_gather_body(val_hbm, idx_hbm, c0, c1, c2, c3, c4, c5, c6, c7,
                    out_hbm, idx_v, cid_v, rows_v, sem):
    wid = lax.axis_index("s") * NC + lax.axis_index("c")
    base = wid * PER_W
    cols = (c0, c1, c2, c3, c4, c5, c6, c7)
    pltpu.sync_copy(idx_hbm.at[pl.ds(base, PER_W)], idx_v)
    pltpu.async_copy(val_hbm.at[idx_v], cid_v, sem).wait()
    copies = [pltpu.async_copy(cols[e].at[cid_v],
                               rows_v.at[pl.ds(e * PER_W, PER_W)], sem)
              for e in range(EMB)]
    for c in copies:
        c.wait()
    for e in range(EMB):
        pltpu.sync_copy(rows_v.at[pl.ds(e * PER_W, PER_W)],
                        out_hbm.at[pl.ds(e * N_IDS + base, PER_W)])


def _sc_gather(ft_sparse_val, ft_sparse_idx, emb_table):
    mesh = plsc.VectorSubcoreMesh(core_axis_name="c", subcore_axis_name="s")
    fn = pl.kernel(
        _sc_gather_body, mesh=mesh,
        out_type=jax.ShapeDtypeStruct((EMB * N_IDS,), jnp.float32),
        scratch_types=[
            pltpu.VMEM((PER_W,), jnp.int32),
            pltpu.VMEM((PER_W,), jnp.int32),
            pltpu.VMEM((EMB * PER_W,), jnp.float32),
            pltpu.SemaphoreType.DMA,
        ],
    )
    cols = _extract_cols(emb_table)
    return fn(ft_sparse_val, ft_sparse_idx, *cols)


def _mlp_body(x_ref, w0_ref, b0_ref, w1_ref, b1_ref, w2_ref, b2_ref, y_ref,
              logits_ref, predict_ref, loss_ref):
    x = x_ref[...]
    h = jnp.dot(x, w0_ref[...], preferred_element_type=jnp.float32)
    h = jnp.maximum(h + b0_ref[...], 0.0)
    h = jnp.dot(h, w1_ref[...], preferred_element_type=jnp.float32)
    h = jnp.maximum(h + b1_ref[...], 0.0)
    z = jnp.dot(h, w2_ref[...], preferred_element_type=jnp.float32) + b2_ref[...]
    logits_ref[...] = z
    predict_ref[...] = jax.nn.sigmoid(z)
    y = y_ref[...]
    loss_vec = jnp.maximum(z, 0.0) - z * y + jnp.log1p(jnp.exp(-jnp.abs(z)))
    loss_ref[0, 0] = jnp.sum(loss_vec) * (1.0 / BATCH)


def _mlp(x, W0, b0, W1, b1, W2, b2, labels):
    logits, predict, loss = pl.pallas_call(
        _mlp_body,
        out_shape=(
            jax.ShapeDtypeStruct((BATCH, 1), jnp.float32),
            jax.ShapeDtypeStruct((BATCH, 1), jnp.float32),
            jax.ShapeDtypeStruct((1, 1), jnp.float32),
        ),
        out_specs=(
            pl.BlockSpec(memory_space=pltpu.VMEM),
            pl.BlockSpec(memory_space=pltpu.VMEM),
            pl.BlockSpec(memory_space=pltpu.SMEM),
        ),
    )(x, W0, b0.reshape(1, -1), W1, b1.reshape(1, -1), W2,
      b2.reshape(1, 1), labels)
    return logits, predict, loss[0, 0]


def kernel(labels, ft_sparse_val, ft_sparse_idx, emb_table, W0, b0, W1, b1,
           W2, b2):
    out1d = _sc_gather(ft_sparse_val, ft_sparse_idx, emb_table)
    x = (out1d.reshape(EMB, BATCH, FEATURE_NUM).transpose(1, 2, 0)
         .reshape(BATCH, FEATURE_NUM * EMB))
    logits, predict, loss = _mlp(x, W0, b0, W1, b1, W2, b2, labels)
    size = jnp.array(emb_table.shape[0], dtype=jnp.int32)
    return (labels, logits, predict, loss, size)
